# baseline (speedup 1.0000x reference)
import functools

import jax
import jax.numpy as jnp
from jax import lax
from jax.experimental import pallas as pl
from jax.experimental.pallas import tpu as pltpu

MASK_VALUE = -1000000000.0
K = 512
LABEL_SMOOTHING = 0.15
T1 = 0.8
T2 = 1.2
B, N, D, H2 = 64, 4096, 256, 128
TN = 128


def _main_body(att_ref, emb_ref, ast_ref, msk_ref, w1_ref, b1_ref, w2_ref,
               b2_ref, res_ref):
    attb = att_ref[...].astype(jnp.bfloat16)
    w2b = w2_ref[...].astype(jnp.bfloat16)
    att_j = jnp.dot(attb, w2b, preferred_element_type=jnp.float32)
    att_j = att_j + b2_ref[...][None, :]
    embb = emb_ref[...].reshape(B * TN, D).astype(jnp.bfloat16)
    w1b = w1_ref[...].astype(jnp.bfloat16)
    cand3 = jnp.dot(embb, w1b,
                    preferred_element_type=jnp.float32).reshape(B, TN, H2)
    cand3 = cand3 + b1_ref[...][None, None, :]
    cand = jnp.sum(ast_ref[...][:, None, :] * cand3, axis=-1)
    cand = cand / jnp.sqrt(jnp.float32(H2))
    r = att_j * cand
    res_ref[...] = jnp.where(msk_ref[...] == 1, MASK_VALUE, r)


def _greater(va, ia, vb, ib):
    return (va > vb) | ((va == vb) & (ia < ib))


def _cmpx(v, i, pos, j, desc_region):
    bitj0 = (pos & j) == 0
    pv = jnp.where(bitj0, jnp.roll(v, -j, axis=-1), jnp.roll(v, j, axis=-1))
    pi = jnp.where(bitj0, jnp.roll(i, -j, axis=-1), jnp.roll(i, j, axis=-1))
    g = _greater(v, i, pv, pi)
    take_mine = g == (bitj0 == desc_region)
    return jnp.where(take_mine, v, pv), jnp.where(take_mine, i, pi)


def _rev(x, pos):
    j = 1
    while j < K:
        bitj0 = (pos & j) == 0
        x = jnp.where(bitj0, jnp.roll(x, -j, axis=-1), jnp.roll(x, j, axis=-1))
        j *= 2
    return x


def _exp_t2(u):
    v = 1.0 + (1.0 - T2) * u
    vs = jnp.where(v > 0, v, 1.0)
    v2 = vs * vs
    v5 = v2 * v2 * vs
    return jnp.where(v > 0, 1.0 / v5, 0.0)


def _powf(x, p):
    return jnp.exp(p * jnp.log(x))


def _post_body(res_ref, att_ref, score_ref, tv_ref, ti_ref, nll_ref):
    r = res_ref[...]
    m = jnp.max(r, axis=-1, keepdims=True)
    e = jnp.exp(r - m)
    s = jnp.sum(e, axis=-1, keepdims=True)
    score = e / s
    score_ref[...] = score

    v = score.reshape(B, 8, K)
    i = (lax.broadcasted_iota(jnp.int32, (B, 8, K), 1) * K
         + lax.broadcasted_iota(jnp.int32, (B, 8, K), 2))
    pos = lax.broadcasted_iota(jnp.int32, (B, 8, K), 2)
    k = 2
    while k <= K:
        desc_region = (pos & k) == 0
        j = k // 2
        while j >= 1:
            v, i = _cmpx(v, i, pos, j, desc_region)
            j //= 2
        k *= 2
    c = 8
    while c > 1:
        va = v.reshape(B, c // 2, 2, K)[:, :, 0, :]
        vb = v.reshape(B, c // 2, 2, K)[:, :, 1, :]
        ia = i.reshape(B, c // 2, 2, K)[:, :, 0, :]
        ib = i.reshape(B, c // 2, 2, K)[:, :, 1, :]
        posc = lax.broadcasted_iota(jnp.int32, (B, c // 2, K), 2)
        vbr = _rev(vb, posc)
        ibr = _rev(ib, posc)
        g = _greater(va, ia, vbr, ibr)
        v = jnp.where(g, va, vbr)
        i = jnp.where(g, ia, ibr)
        j = K // 2
        while j >= 1:
            v, i = _cmpx(v, i, posc, j, True)
            j //= 2
        c //= 2
    tv_ref[...] = v.reshape(B, K)
    ti_ref[...] = i.reshape(B, K)

    label = att_ref[...] / 10.0
    label = label * (1.0 - LABEL_SMOOTHING) + LABEL_SMOOTHING / N
    a0 = r - m
    normalized = a0
    for _ in range(5):
        lp = jnp.sum(_exp_t2(normalized), axis=-1, keepdims=True)
        normalized = a0 * _powf(lp, 1.0 - T2)
    lp = jnp.sum(_exp_t2(normalized), axis=-1, keepdims=True)
    norm = -(_powf(1.0 / lp, 1.0 - T2) - 1.0) / (1.0 - T2) + m
    probs = _exp_t2(r - norm)
    log_t_label = (_powf(label + 1e-10, 1.0 - T1) - 1.0) / (1.0 - T1)
    log_t_probs = (_powf(probs + 1e-10, 1.0 - T1) - 1.0) / (1.0 - T1)
    loss = (label * (log_t_label - log_t_probs)
            - (_powf(label, 2.0 - T1) - _powf(probs, 2.0 - T1)) / (2.0 - T1))
    nll_ref[...] = jnp.sum(loss).reshape(1, 1)


@jax.jit
def kernel(embedding, agent_state, attention, mask_int, W1, b1, W2, b2):
    grid = (N // TN,)
    result = pl.pallas_call(
        _main_body,
        grid=grid,
        in_specs=[
            pl.BlockSpec((B, N), lambda j: (0, 0)),
            pl.BlockSpec((B, TN, D), lambda j: (0, j, 0)),
            pl.BlockSpec((B, H2), lambda j: (0, 0)),
            pl.BlockSpec((B, TN), lambda j: (0, j)),
            pl.BlockSpec((D, H2), lambda j: (0, 0)),
            pl.BlockSpec((H2,), lambda j: (0,)),
            pl.BlockSpec((N, TN), lambda j: (0, j)),
            pl.BlockSpec((TN,), lambda j: (j,)),
        ],
        out_specs=pl.BlockSpec((B, TN), lambda j: (0, j)),
        out_shape=jax.ShapeDtypeStruct((B, N), jnp.float32),
    )(attention, embedding, agent_state, mask_int, W1, b1, W2, b2)

    score, top_vals, top_idx, nll = pl.pallas_call(
        _post_body,
        out_shape=[
            jax.ShapeDtypeStruct((B, N), jnp.float32),
            jax.ShapeDtypeStruct((B, K), jnp.float32),
            jax.ShapeDtypeStruct((B, K), jnp.int32),
            jax.ShapeDtypeStruct((1, 1), jnp.float32),
        ],
    )(result, attention)
    return score, top_vals, top_idx, nll.reshape(())
